# baseline (speedup 1.0000x reference)
import functools

import jax
import jax.numpy as jnp
from jax import lax
from jax.experimental import pallas as pl
from jax.experimental.pallas import tpu as pltpu
from jax.experimental.pallas import tpu_sc as plsc

_C = 64
_SLOTS = 68
_NW = 32
_WINW = 40
_NG = 5
_LANES = 16


def _sc_span_means(hidden_g, idxg, wts, n_spans, d):
    spw = n_spans // _NW
    mesh = plsc.VectorSubcoreMesh(core_axis_name="c", subcore_axis_name="s")

    @functools.partial(
        pl.kernel,
        out_type=jax.ShapeDtypeStruct((_NW, spw, d), jnp.float32),
        mesh=mesh,
        scratch_types=[
            pltpu.VMEM((spw, _NG), jnp.int32),
            pltpu.VMEM((spw * _NG, 128), jnp.float32),
            pltpu.VMEM((_NG, 8, d), jnp.float32),
            pltpu.VMEM((_NG, 8, d), jnp.float32),
            pltpu.VMEM((spw, d), jnp.float32),
            pltpu.SemaphoreType.DMA,
            pltpu.SemaphoreType.DMA,
        ],
    )
    def body(hid_hbm, idx_hbm, wts_hbm, out_hbm, idx_v, wts_v, rows0_v,
             rows1_v, sums_v, sem0, sem1):
        wid = lax.axis_index("c") * 16 + lax.axis_index("s")
        pltpu.sync_copy(idx_hbm.at[wid], idx_v)
        pltpu.sync_copy(wts_hbm.at[wid], wts_v)
        rows = (rows0_v, rows1_v)
        sems = (sem0, sem1)

        for b0 in range(2):
            pltpu.async_copy(hid_hbm.at[idx_v.at[b0]], rows[b0], sems[b0])

        _KC = 16
        n_pass = d // (_KC * _LANES)

        def pair_body(g, carry):
            for b in range(2):
                t = g * 2 + b
                pltpu.make_async_copy(
                    hid_hbm.at[idx_v.at[t]], rows[b], sems[b]
                ).wait()
                buf = rows[b]

                for p in range(n_pass):
                    accs = tuple(
                        jnp.zeros((_LANES,), jnp.float32) for _ in range(_KC)
                    )
                    for gg in range(_NG):
                        def row_body(j8, a, _p=p, _gg=gg, _buf=buf, _t=t):
                            w = wts_v[_t * _NG + _gg, pl.ds(j8 * _LANES, _LANES)]
                            return tuple(
                                a[k]
                                + _buf[_gg, j8,
                                       pl.ds(_p * _KC * _LANES + k * _LANES,
                                             _LANES)] * w
                                for k in range(_KC)
                            )

                        accs = lax.fori_loop(0, 8, row_body, accs)
                    for k in range(_KC):
                        sums_v[t, pl.ds(p * _KC * _LANES + k * _LANES,
                                        _LANES)] = accs[k]

                @pl.when(t + 2 < spw)
                def _refill(_b=b):
                    pltpu.async_copy(
                        hid_hbm.at[idx_v.at[t + 2]], rows[_b], sems[_b]
                    )

            return carry

        lax.fori_loop(0, spw // 2, pair_body, 0)
        pltpu.sync_copy(sums_v, out_hbm.at[wid])

    return body(hidden_g, idxg, wts)


def _tc_finish(means, n_valid):
    b, slots, d = means.shape

    def body(means_ref, nv_ref, out_ref):
        cand = means_ref[:, :_C, :]
        num = means_ref[:, _C:_C + 1, :]
        dots = jnp.sum(cand * num, axis=-1)
        cid = lax.broadcasted_iota(jnp.int32, (b, _C), 1)
        out_ref[:] = jnp.where(cid < nv_ref[:], dots, 0.0)

    return pl.pallas_call(
        body,
        out_shape=jax.ShapeDtypeStruct((b, _C), jnp.float32),
    )(means, n_valid)


def kernel(last_hidden, cand_starts, cand_lens, num_starts, num_lens, n_valid):
    B, S, D = last_hidden.shape
    n_spans = B * _SLOTS

    cand_starts = cand_starts.astype(jnp.int32)
    cand_lens = cand_lens.astype(jnp.int32)
    num_starts = num_starts.astype(jnp.int32)
    num_lens = num_lens.astype(jnp.int32)

    pad = _SLOTS - _C - 1
    starts = jnp.concatenate(
        [cand_starts, num_starts[:, None], jnp.zeros((B, pad), jnp.int32)], axis=1
    )
    lens = jnp.concatenate(
        [cand_lens, num_lens[:, None], jnp.ones((B, pad), jnp.int32)], axis=1
    )
    lens = jnp.maximum(lens, 1)
    starts = jnp.clip(starts, 0, S - 1)
    ends = jnp.clip(starts + lens, 1, S)
    eff = ends - starts

    flat = starts + jnp.arange(B, dtype=jnp.int32)[:, None] * S
    wg = jnp.minimum(
        flat // 8,
        jnp.arange(B, dtype=jnp.int32)[:, None] * (S // 8) + (S // 8 - _NG),
    )
    idxg = (
        wg[:, :, None] + jnp.arange(_NG, dtype=jnp.int32)[None, None, :]
    ).astype(jnp.int32)
    idxg = idxg.reshape(_NW, n_spans // _NW, _NG)

    off = flat - wg * 8
    j = jnp.arange(_WINW, dtype=jnp.int32)
    inspan = (j[None, None, :] >= off[:, :, None]) & (
        j[None, None, :] < (off + eff)[:, :, None]
    )
    w = jnp.where(inspan, 1.0 / eff[:, :, None].astype(jnp.float32), 0.0)
    wts = jnp.broadcast_to(
        w[:, :, :, None], (B, _SLOTS, _WINW, _LANES)
    ).reshape(_NW, (n_spans // _NW) * _NG, 128).astype(jnp.float32)

    hidden_g = last_hidden.reshape(B * S // 8, 8, D)
    means = _sc_span_means(hidden_g, idxg, wts, n_spans, D).reshape(B, _SLOTS, D)

    return _tc_finish(means, n_valid.astype(jnp.int32)[:, None])


# speedup vs baseline: 1.0075x; 1.0075x over previous
import functools

import jax
import jax.numpy as jnp
from jax import lax
from jax.experimental import pallas as pl
from jax.experimental.pallas import tpu as pltpu
from jax.experimental.pallas import tpu_sc as plsc

_C = 64
_SLOTS = 68
_NW = 32
_WINW = 40
_NG = 5
_LANES = 16


def _sc_span_means(n_spans, d):
    spw = n_spans // _NW
    mesh = plsc.VectorSubcoreMesh(core_axis_name="c", subcore_axis_name="s")

    @functools.partial(
        pl.kernel,
        out_type=jax.ShapeDtypeStruct((_NW, spw, d), jnp.float32),
        mesh=mesh,
        scratch_types=[
            pltpu.VMEM((spw, _NG), jnp.int32),
            pltpu.VMEM((spw * _NG, 128), jnp.float32),
            pltpu.VMEM((_NG, 8, d), jnp.float32),
            pltpu.VMEM((_NG, 8, d), jnp.float32),
            pltpu.VMEM((spw, d), jnp.float32),
            pltpu.SemaphoreType.DMA,
            pltpu.SemaphoreType.DMA,
        ],
    )
    def body(hid_hbm, idx_hbm, wts_hbm, out_hbm, idx_v, wts_v,
             rows0_v, rows1_v, sums_v, sem0, sem1):
        wid = lax.axis_index("c") * 16 + lax.axis_index("s")
        pltpu.sync_copy(idx_hbm.at[wid], idx_v)
        pltpu.sync_copy(wts_hbm.at[wid], wts_v)
        rows = (rows0_v, rows1_v)
        sems = (sem0, sem1)

        for b0 in range(2):
            pltpu.async_copy(hid_hbm.at[idx_v.at[b0]], rows[b0], sems[b0])

        _KC = 16
        n_pass = d // (_KC * _LANES)

        def pair_body(g, carry):
            for b in range(2):
                t = g * 2 + b
                pltpu.make_async_copy(
                    hid_hbm.at[idx_v.at[t]], rows[b], sems[b]
                ).wait()
                buf = rows[b]

                for p in range(n_pass):
                    accs = tuple(
                        jnp.zeros((_LANES,), jnp.float32) for _ in range(_KC)
                    )
                    for gg in range(_NG):
                        def row_body(j8, a, _p=p, _gg=gg, _buf=buf, _t=t):
                            w = wts_v[_t * _NG + _gg, pl.ds(j8 * _LANES, _LANES)]
                            return tuple(
                                a[k]
                                + _buf[_gg, j8,
                                       pl.ds(_p * _KC * _LANES + k * _LANES,
                                             _LANES)] * w
                                for k in range(_KC)
                            )

                        accs = lax.fori_loop(0, 8, row_body, accs)
                    for k in range(_KC):
                        sums_v[t, pl.ds(p * _KC * _LANES + k * _LANES,
                                        _LANES)] = accs[k]

                @pl.when(t + 2 < spw)
                def _refill(_b=b):
                    pltpu.async_copy(
                        hid_hbm.at[idx_v.at[t + 2]], rows[_b], sems[_b]
                    )

            return carry

        lax.fori_loop(0, spw // 2, pair_body, 0)
        pltpu.sync_copy(sums_v, out_hbm.at[wid])

    return body


def _tc_finish(means, n_valid):
    b, slots, d = means.shape

    def body(means_ref, nv_ref, out_ref):
        cand = means_ref[:, :_C, :]
        num = means_ref[:, _C:_C + 1, :]
        dots = jnp.sum(cand * num, axis=-1)
        cid = lax.broadcasted_iota(jnp.int32, (b, _C), 1)
        out_ref[:] = jnp.where(cid < nv_ref[:], dots, 0.0)

    return pl.pallas_call(
        body,
        out_shape=jax.ShapeDtypeStruct((b, _C), jnp.float32),
    )(means, n_valid)


def kernel(last_hidden, cand_starts, cand_lens, num_starts, num_lens, n_valid):
    B, S, D = last_hidden.shape
    n_spans = B * _SLOTS

    cand_starts = cand_starts.astype(jnp.int32)
    cand_lens = cand_lens.astype(jnp.int32)
    num_starts = num_starts.astype(jnp.int32)
    num_lens = num_lens.astype(jnp.int32)

    pad = _SLOTS - _C - 1
    starts = jnp.concatenate(
        [cand_starts, num_starts[:, None], jnp.zeros((B, pad), jnp.int32)], axis=1
    )
    lens = jnp.concatenate(
        [cand_lens, num_lens[:, None], jnp.ones((B, pad), jnp.int32)], axis=1
    )
    lens = jnp.maximum(lens, 1)
    starts = jnp.clip(starts, 0, S - 1)
    ends = jnp.clip(starts + lens, 1, S)
    eff = ends - starts

    flat = starts + jnp.arange(B, dtype=jnp.int32)[:, None] * S
    wg = jnp.minimum(
        flat // 8,
        jnp.arange(B, dtype=jnp.int32)[:, None] * (S // 8) + (S // 8 - _NG),
    )
    idxg = (
        wg[:, :, None] + jnp.arange(_NG, dtype=jnp.int32)[None, None, :]
    ).astype(jnp.int32)
    idxg = idxg.reshape(_NW, n_spans // _NW, _NG)

    off = flat - wg * 8
    j = jnp.arange(_WINW, dtype=jnp.int32)
    inspan = (j[None, None, :] >= off[:, :, None]) & (
        j[None, None, :] < (off + eff)[:, :, None]
    )
    w = jnp.where(inspan, 1.0 / eff[:, :, None].astype(jnp.float32), 0.0)
    wts = jnp.broadcast_to(
        w[:, :, :, None], (B, _SLOTS, _WINW, _LANES)
    ).reshape(_NW, (n_spans // _NW) * _NG, 128).astype(jnp.float32)

    hidden_g = last_hidden.reshape(B * S // 8, 8, D)
    means = _sc_span_means(n_spans, D)(hidden_g, idxg, wts)
    means = means.reshape(B, _SLOTS, D)

    return _tc_finish(means, n_valid.astype(jnp.int32)[:, None])
